# baseline (speedup 1.0000x reference)
import functools

import jax
import jax.numpy as jnp
from jax import lax
from jax.experimental import pallas as pl
from jax.experimental.pallas import tpu as pltpu
from jax.experimental.pallas import tpu_sc as plsc

B = 16384
BAG = 50
D = 64
V = 1000000

HALF = 512000
VT = 2 * HALF

NC = 2
NS = 16
NW = NC * NS

BAGS_PER_W = B // NW
ROWS_PER_W = BAGS_PER_W * BAG
CHUNK_BAGS = 8
CHUNK_ROWS = CHUNK_BAGS * BAG
N_CHUNKS = BAGS_PER_W // CHUNK_BAGS
SUB = 80
N_SUB = CHUNK_ROWS // SUB

VB2 = 25600


def _tp_body(in1_ref, in2_ref, out_ref):
    stacked = jnp.concatenate([in1_ref[...], in2_ref[...]], axis=0)
    out_ref[...] = stacked.T.astype(jnp.bfloat16)


def _tc_transpose(wt):
    return pl.pallas_call(
        _tp_body,
        grid=(HALF // VB2,),
        in_specs=[
            pl.BlockSpec((D, VB2), lambda i: (0, i)),
            pl.BlockSpec(
                (D, VB2),
                lambda i: (0, jnp.minimum(i + HALF // VB2, (V - 1) // VB2)),
            ),
        ],
        out_specs=pl.BlockSpec((VB2, 2 * D), lambda i: (i, 0)),
        out_shape=jax.ShapeDtypeStruct((HALF, 2 * D), jnp.bfloat16),
    )(wt, wt)


def _fire(w_hbm, idx_v, buf, sem, g):
    base = g * CHUNK_ROWS
    for s in range(N_SUB):
        pltpu.async_copy(
            w_hbm.at[idx_v.at[pl.ds(base + s * SUB, SUB)]],
            buf.at[pl.ds(s * SUB, SUB)],
            sem,
        )


def _drain(w_hbm, buf, sem):
    pltpu.make_async_copy(w_hbm.at[pl.ds(0, CHUNK_ROWS)], buf, sem).wait()


def _accumulate(buf, ob, bias_vecs, col_idx):
    def bag_body(bb, carry):
        row0 = bb * BAG

        def body(j, accs):
            r = row0 + j
            res = []
            for k in range(2):
                x = buf[r, pl.ds(32 * k, 32)]
                a, b = plsc.unpack(x, format=plsc.PackFormat.INTERLEAVED)
                res.append(accs[2 * k] + a)
                res.append(accs[2 * k + 1] + b)
            return tuple(res)

        accs = lax.fori_loop(0, BAG, body, bias_vecs, unroll=10)
        row_idx = jnp.full((16,), bb, jnp.int32)
        for k in range(4):
            plsc.store_scatter(ob, [row_idx, col_idx[k]], accs[k])
        return carry

    lax.fori_loop(0, CHUNK_BAGS, bag_body, 0)


def _remap_chunk(idx_v, g):
    n = CHUNK_ROWS // 16
    base = g * n

    def body(i, carry):
        off = 16 * (base + i)
        v = idx_v[pl.ds(off, 16)]
        two_v = v + v
        j = jnp.where(v < HALF, two_v, two_v - (VT - 1))
        idx_v[pl.ds(off, 16)] = j
        return carry

    lax.fori_loop(0, n, body, 0, unroll=5)


def _sc_body(idx_hbm, w_hbm, bias_hbm, out_hbm,
             idx_v, rows0, rows1, rows2, rows3,
             ob0, ob1, ob2, ob3, bias_v,
             sem0, sem1, sem2, sem3, osem):
    wid = lax.axis_index("s") * NC + lax.axis_index("c")
    bag0 = wid * BAGS_PER_W

    pltpu.sync_copy(bias_hbm, bias_v)
    pltpu.sync_copy(idx_hbm.at[pl.ds(wid * ROWS_PER_W, ROWS_PER_W)], idx_v)

    lanes = lax.iota(jnp.int32, 16)
    col_idx = tuple(32 * k + 2 * lanes + p for k in range(2) for p in range(2))
    bias_vecs = tuple(plsc.load_gather(bias_v, [ci]) for ci in col_idx)
    bufs = (rows0, rows1, rows2, rows3)
    sems = (sem0, sem1, sem2, sem3)
    obs = (ob0, ob1, ob2, ob3)

    def wait_one_out():
        pltpu.make_async_copy(
            out_hbm.at[pl.ds(0, CHUNK_BAGS)], obs[0], osem).wait()

    def step(g, b, fire_next, out_wait):
        if out_wait:
            wait_one_out()
        if fire_next:
            _remap_chunk(idx_v, g + 3)
            _fire(w_hbm, idx_v, bufs[(b + 3) % 4], sems[(b + 3) % 4], g + 3)
        _drain(w_hbm, bufs[b], sems[b])
        _accumulate(bufs[b], obs[b], bias_vecs, col_idx)
        pltpu.async_copy(
            obs[b],
            out_hbm.at[pl.ds(bag0 + g * CHUNK_BAGS, CHUNK_BAGS)],
            osem,
        )

    for g in range(3):
        _remap_chunk(idx_v, g)
        _fire(w_hbm, idx_v, bufs[g], sems[g], g)

    for b in range(4):
        step(b, b, fire_next=True, out_wait=False)

    def chunk_quad(i, carry):
        for b in range(4):
            g = 4 * (i + 1) + b
            step(g, b, fire_next=True, out_wait=True)
        return carry

    lax.fori_loop(0, (N_CHUNKS - 8) // 4, chunk_quad, 0)

    step(N_CHUNKS - 4, 0, fire_next=True, out_wait=True)
    for b in range(1, 4):
        step(N_CHUNKS - 4 + b, b, fire_next=False, out_wait=True)

    for _ in range(4):
        wait_one_out()


@jax.jit
def _embed_bag(indices, w, bias):
    w_packed = _tc_transpose(w.T)
    w_rows = w_packed.reshape(VT, D)
    mesh = plsc.VectorSubcoreMesh(core_axis_name="c", subcore_axis_name="s")
    run = pl.kernel(
        _sc_body,
        out_type=jax.ShapeDtypeStruct((B, D), jnp.float32),
        mesh=mesh,
        scratch_types=[
            pltpu.VMEM((ROWS_PER_W,), jnp.int32),
            pltpu.VMEM((CHUNK_ROWS, D), jnp.bfloat16),
            pltpu.VMEM((CHUNK_ROWS, D), jnp.bfloat16),
            pltpu.VMEM((CHUNK_ROWS, D), jnp.bfloat16),
            pltpu.VMEM((CHUNK_ROWS, D), jnp.bfloat16),
            pltpu.VMEM((CHUNK_BAGS, D), jnp.float32),
            pltpu.VMEM((CHUNK_BAGS, D), jnp.float32),
            pltpu.VMEM((CHUNK_BAGS, D), jnp.float32),
            pltpu.VMEM((CHUNK_BAGS, D), jnp.float32),
            pltpu.VMEM((D,), jnp.float32),
            pltpu.SemaphoreType.DMA,
            pltpu.SemaphoreType.DMA,
            pltpu.SemaphoreType.DMA,
            pltpu.SemaphoreType.DMA,
            pltpu.SemaphoreType.DMA,
        ],
        compiler_params=pltpu.CompilerParams(
            use_tc_tiling_on_sc=False, needs_layout_passes=False),
    )
    return run(indices, w_rows, bias)


def kernel(indices, offsets, W, bias):
    del offsets
    return _embed_bag(indices.astype(jnp.int32), W, bias)


# speedup vs baseline: 2.3418x; 2.3418x over previous
import functools

import jax
import jax.numpy as jnp
from jax import lax
from jax.experimental import pallas as pl
from jax.experimental.pallas import tpu as pltpu
from jax.experimental.pallas import tpu_sc as plsc

B = 16384
BAG = 50
D = 64
V = 1000000

HALF = 512000
VT = 2 * HALF

NC = 2
NS = 16
NW = NC * NS

BAGS_PER_W = B // NW
ROWS_PER_W = BAGS_PER_W * BAG
CHUNK_BAGS = 8
CHUNK_ROWS = CHUNK_BAGS * BAG
N_CHUNKS = BAGS_PER_W // CHUNK_BAGS
SUB = 80
N_SUB = CHUNK_ROWS // SUB

VB2 = 25600


def _tp_body(in1_ref, in2_ref, out_ref):
    stacked = jnp.concatenate([in1_ref[...], in2_ref[...]], axis=0)
    out_ref[...] = stacked.T


def _tc_transpose(wt):
    return pl.pallas_call(
        _tp_body,
        grid=(HALF // VB2,),
        in_specs=[
            pl.BlockSpec((D, VB2), lambda i: (0, i)),
            pl.BlockSpec(
                (D, VB2),
                lambda i: (0, jnp.minimum(i + HALF // VB2, (V - 1) // VB2)),
            ),
        ],
        out_specs=pl.BlockSpec((VB2, 2 * D), lambda i: (i, 0)),
        out_shape=jax.ShapeDtypeStruct((HALF, 2 * D), jnp.float32),
    )(wt, wt)


def _fire(w_hbm, idx_v, buf, sem, g):
    base = g * CHUNK_ROWS
    for s in range(N_SUB):
        pltpu.async_copy(
            w_hbm.at[idx_v.at[pl.ds(base + s * SUB, SUB)]],
            buf.at[pl.ds(s * SUB, SUB)],
            sem,
        )


def _drain(w_hbm, buf, sem):
    pltpu.make_async_copy(w_hbm.at[pl.ds(0, CHUNK_ROWS)], buf, sem).wait()


def _accumulate(buf, ob, bias_vecs):
    def bag_body(bb, carry):
        row0 = bb * BAG

        def body(j, accs):
            r = row0 + j
            return tuple(
                accs[k] + buf[r, pl.ds(16 * k, 16)] for k in range(4)
            )

        accs = lax.fori_loop(0, BAG, body, bias_vecs, unroll=10)
        for k in range(4):
            ob[bb, pl.ds(16 * k, 16)] = accs[k]
        return carry

    lax.fori_loop(0, CHUNK_BAGS, bag_body, 0)


def _remap_chunk(idx_v, g):
    n = CHUNK_ROWS // 16
    base = g * n

    def body(i, carry):
        off = 16 * (base + i)
        v = idx_v[pl.ds(off, 16)]
        two_v = v + v
        j = jnp.where(v < HALF, two_v, two_v - (VT - 1))
        idx_v[pl.ds(off, 16)] = j
        return carry

    lax.fori_loop(0, n, body, 0, unroll=5)


def _sc_body(idx_hbm, w_hbm, bias_hbm, out_hbm,
             idx_v, rows0, rows1, rows2, rows3,
             ob0, ob1, ob2, ob3, bias_v,
             sem0, sem1, sem2, sem3, osem):
    wid = lax.axis_index("s") * NC + lax.axis_index("c")
    bag0 = wid * BAGS_PER_W

    pltpu.sync_copy(bias_hbm, bias_v)
    pltpu.sync_copy(idx_hbm.at[pl.ds(wid * ROWS_PER_W, ROWS_PER_W)], idx_v)

    bias_vecs = tuple(bias_v[pl.ds(16 * k, 16)] for k in range(4))
    bufs = (rows0, rows1, rows2, rows3)
    sems = (sem0, sem1, sem2, sem3)
    obs = (ob0, ob1, ob2, ob3)

    def wait_one_out():
        pltpu.make_async_copy(
            out_hbm.at[pl.ds(0, CHUNK_BAGS)], obs[0], osem).wait()

    def step(g, b, fire_next, out_wait):
        if out_wait:
            wait_one_out()
        if fire_next:
            _remap_chunk(idx_v, g + 3)
            _fire(w_hbm, idx_v, bufs[(b + 3) % 4], sems[(b + 3) % 4], g + 3)
        _drain(w_hbm, bufs[b], sems[b])
        _accumulate(bufs[b], obs[b], bias_vecs)
        pltpu.async_copy(
            obs[b],
            out_hbm.at[pl.ds(bag0 + g * CHUNK_BAGS, CHUNK_BAGS)],
            osem,
        )

    for g in range(3):
        _remap_chunk(idx_v, g)
        _fire(w_hbm, idx_v, bufs[g], sems[g], g)

    for b in range(4):
        step(b, b, fire_next=True, out_wait=False)

    def chunk_quad(i, carry):
        for b in range(4):
            g = 4 * (i + 1) + b
            step(g, b, fire_next=True, out_wait=True)
        return carry

    lax.fori_loop(0, (N_CHUNKS - 8) // 4, chunk_quad, 0)

    step(N_CHUNKS - 4, 0, fire_next=True, out_wait=True)
    for b in range(1, 4):
        step(N_CHUNKS - 4 + b, b, fire_next=False, out_wait=True)

    for _ in range(4):
        wait_one_out()


@jax.jit
def _embed_bag(indices, w, bias):
    w_packed = _tc_transpose(w.T)
    w_rows = w_packed.reshape(VT, D)
    mesh = plsc.VectorSubcoreMesh(core_axis_name="c", subcore_axis_name="s")
    run = pl.kernel(
        _sc_body,
        out_type=jax.ShapeDtypeStruct((B, D), jnp.float32),
        mesh=mesh,
        scratch_types=[
            pltpu.VMEM((ROWS_PER_W,), jnp.int32),
            pltpu.VMEM((CHUNK_ROWS, D), jnp.float32),
            pltpu.VMEM((CHUNK_ROWS, D), jnp.float32),
            pltpu.VMEM((CHUNK_ROWS, D), jnp.float32),
            pltpu.VMEM((CHUNK_ROWS, D), jnp.float32),
            pltpu.VMEM((CHUNK_BAGS, D), jnp.float32),
            pltpu.VMEM((CHUNK_BAGS, D), jnp.float32),
            pltpu.VMEM((CHUNK_BAGS, D), jnp.float32),
            pltpu.VMEM((CHUNK_BAGS, D), jnp.float32),
            pltpu.VMEM((D,), jnp.float32),
            pltpu.SemaphoreType.DMA,
            pltpu.SemaphoreType.DMA,
            pltpu.SemaphoreType.DMA,
            pltpu.SemaphoreType.DMA,
            pltpu.SemaphoreType.DMA,
        ],
        compiler_params=pltpu.CompilerParams(use_tc_tiling_on_sc=False),
    )
    return run(indices, w_rows, bias)


def kernel(indices, offsets, W, bias):
    del offsets
    return _embed_bag(indices.astype(jnp.int32), W, bias)
